# baseline (speedup 1.0000x reference)
import functools

import jax
import jax.numpy as jnp
from jax.experimental import pallas as pl
from jax.experimental.pallas import tpu as pltpu


def _gelu(x):
    return x * jax.nn.sigmoid(1.5957691216057308 * x +
                              0.07135481627362622 * (x * x * x))


def _fused_block(x_ref, wpre_ref, wres16_ref, wrt_ref, w116_ref, w216_ref,
                 wpost16_ref, out_ref, carry_ref, *, lb):
    i = pl.program_id(0)
    f32 = jnp.float32
    bf16 = jnp.bfloat16

    @pl.when(i == 0)
    def _():
        carry_ref[...] = jnp.zeros_like(carry_ref)

    x = x_ref[0]
    h = jax.nn.gelu(jnp.dot(x, wpre_ref[...], preferred_element_type=f32))
    h16 = h.astype(bf16)
    res = jnp.dot(h16, wres16_ref[...], preferred_element_type=f32)
    logits = jnp.dot(h, wrt_ref[...], preferred_element_type=f32)

    sb = 256
    row = jax.lax.broadcasted_iota(jnp.int32, (sb, 1), 0)
    rowf = row.astype(f32)
    rowi = jax.lax.broadcasted_iota(jnp.int32, (sb, sb), 0)
    colj = jax.lax.broadcasted_iota(jnp.int32, (sb, sb), 1)
    eye = (rowi == colj).astype(f32)
    colj_glob = jax.lax.broadcasted_iota(jnp.int32, (sb, lb), 1).astype(f32)

    probs_col = jax.nn.sigmoid(logits)

    t16 = jax.nn.gelu(jnp.dot(h16, w116_ref[...],
                              preferred_element_type=f32)).astype(bf16)
    mid = jnp.dot(t16, w216_ref[...], preferred_element_type=f32)
    yg = mid * probs_col

    carry_row = carry_ref[7:8, :]
    prev_max = jnp.full((1, 1), -1.0, f32)
    for s in range(lb // sb):
        probs_s = probs_col[s * sb:(s + 1) * sb]
        mask = (probs_s >= 0.5) | ((row == 0) & (i == 0) & (s == 0))
        c_col = jnp.where(mask, rowf, -1.0)
        c_row = jnp.dot(jnp.ones((1, sb), f32), eye * c_col,
                        preferred_element_type=f32)
        m_mat = jnp.where(colj <= rowi,
                          jnp.broadcast_to(c_row, (sb, sb)), -1.0)
        b_col = jnp.max(m_mat, axis=1, keepdims=True)
        b_col = jnp.where(b_col >= 0.0, b_col + float(s * sb), -1.0)
        b_col = jnp.maximum(b_col, prev_max)
        prev_max = b_col[sb - 1:, :]
        sel = (b_col == colj_glob).astype(f32)
        up_s = jnp.dot(sel, yg, preferred_element_type=f32)
        up_s = up_s + jnp.where(b_col < 0.0, carry_row, 0.0)
        if s == lb // sb - 1:
            carry_ref[...] = up_s[sb - 8:, :]
        fused16 = (res[s * sb:(s + 1) * sb] + up_s).astype(bf16)
        out_ref[0, s * sb:(s + 1) * sb, :] = jax.nn.gelu(
            jnp.dot(fused16, wpost16_ref[...], preferred_element_type=f32))


def kernel(hidden_states, x_pack_kwargs, W_pre, W_res, w_router, W1, W2,
           W_post):
    del x_pack_kwargs
    B, L, D = hidden_states.shape
    d_ff = W1.shape[1]
    lb = 512
    wrt = w_router.reshape(D, 1)
    W_res = W_res.astype(jnp.bfloat16)
    W1 = W1.astype(jnp.bfloat16)
    W2 = W2.astype(jnp.bfloat16)
    W_post = W_post.astype(jnp.bfloat16)

    grid = (L // lb,)
    full = lambda a: pl.BlockSpec(a.shape, lambda i: (0,) * a.ndim)
    out = pl.pallas_call(
        functools.partial(_fused_block, lb=lb),
        grid=grid,
        in_specs=[
            pl.BlockSpec((1, lb, D), lambda i: (0, i, 0)),
            full(W_pre), full(W_res), full(wrt), full(W1), full(W2),
            full(W_post),
        ],
        out_specs=pl.BlockSpec((1, lb, D), lambda i: (0, i, 0)),
        out_shape=jax.ShapeDtypeStruct((B, L, D), jnp.float32),
        scratch_shapes=[
            pltpu.VMEM((8, D), jnp.float32),
        ],
        compiler_params=pltpu.CompilerParams(
            dimension_semantics=("arbitrary",)),
    )(hidden_states, W_pre, W_res, wrt, W1, W2, W_post)
    return out


# speedup vs baseline: 1.1866x; 1.1866x over previous
import functools

import jax
import jax.numpy as jnp
from jax.experimental import pallas as pl
from jax.experimental.pallas import tpu as pltpu


def _fused_block(x_ref, wpre_ref, wres_ref, wrt_ref, w1_ref, w2_ref,
                 wpost_ref, out_ref, carry_ref, w116_ref, w2p16_ref,
                 wrp16_ref, *, lb):
    i = pl.program_id(0)
    f32 = jnp.float32
    bf16 = jnp.bfloat16

    @pl.when(i == 0)
    def _():
        carry_ref[...] = jnp.zeros_like(carry_ref)
        w116_ref[...] = w1_ref[...].astype(bf16)
        wpost16 = wpost_ref[...].astype(bf16)
        w2p16_ref[...] = jnp.dot(w2_ref[...].astype(bf16), wpost16,
                                 preferred_element_type=f32).astype(bf16)
        wrp16_ref[...] = jnp.dot(wres_ref[...].astype(bf16), wpost16,
                                 preferred_element_type=f32).astype(bf16)

    x = x_ref[0]
    h = jax.nn.gelu(jnp.dot(x, wpre_ref[...], preferred_element_type=f32))
    h16 = h.astype(bf16)
    hres = jnp.dot(h16, wrp16_ref[...], preferred_element_type=f32)
    logits = jnp.dot(h, wrt_ref[...], preferred_element_type=f32)

    sb = 256
    row = jax.lax.broadcasted_iota(jnp.int32, (sb, 1), 0)
    rowf = row.astype(f32)
    rowi = jax.lax.broadcasted_iota(jnp.int32, (sb, sb), 0)
    colj = jax.lax.broadcasted_iota(jnp.int32, (sb, sb), 1)
    eye = (rowi == colj).astype(f32)
    colj_glob = jax.lax.broadcasted_iota(jnp.int32, (sb, lb), 1).astype(f32)

    probs_col = jax.nn.sigmoid(logits)

    t16 = jax.nn.gelu(jnp.dot(h16, w116_ref[...],
                              preferred_element_type=f32)).astype(bf16)
    z = jnp.dot(t16, w2p16_ref[...],
                preferred_element_type=f32) * probs_col

    carry_row = carry_ref[7:8, :]
    prev_max = jnp.full((1, 1), -1.0, f32)
    for s in range(lb // sb):
        probs_s = probs_col[s * sb:(s + 1) * sb]
        mask = (probs_s >= 0.5) | ((row == 0) & (i == 0) & (s == 0))
        c_col = jnp.where(mask, rowf, -1.0)
        c_row = jnp.dot(jnp.ones((1, sb), f32), eye * c_col,
                        preferred_element_type=f32)
        m_mat = jnp.where(colj <= rowi,
                          jnp.broadcast_to(c_row, (sb, sb)), -1.0)
        b_col = jnp.max(m_mat, axis=1, keepdims=True)
        b_col = jnp.where(b_col >= 0.0, b_col + float(s * sb), -1.0)
        b_col = jnp.maximum(b_col, prev_max)
        prev_max = b_col[sb - 1:, :]
        sel = (b_col == colj_glob).astype(f32)
        up_s = jnp.dot(sel, z, preferred_element_type=f32)
        up_s = up_s + jnp.where(b_col < 0.0, carry_row, 0.0)
        if s == lb // sb - 1:
            carry_ref[...] = up_s[sb - 8:, :]
        out_ref[0, s * sb:(s + 1) * sb, :] = jax.nn.gelu(
            hres[s * sb:(s + 1) * sb] + up_s)


def kernel(hidden_states, x_pack_kwargs, W_pre, W_res, w_router, W1, W2,
           W_post):
    del x_pack_kwargs
    B, L, D = hidden_states.shape
    d_ff = W1.shape[1]
    lb = 512
    wrt = w_router.reshape(D, 1)

    grid = (L // lb,)
    full = lambda a: pl.BlockSpec(a.shape, lambda i: (0,) * a.ndim)
    out = pl.pallas_call(
        functools.partial(_fused_block, lb=lb),
        grid=grid,
        in_specs=[
            pl.BlockSpec((1, lb, D), lambda i: (0, i, 0)),
            full(W_pre), full(W_res), full(wrt), full(W1), full(W2),
            full(W_post),
        ],
        out_specs=pl.BlockSpec((1, lb, D), lambda i: (0, i, 0)),
        out_shape=jax.ShapeDtypeStruct((B, L, D), jnp.float32),
        scratch_shapes=[
            pltpu.VMEM((8, D), jnp.float32),
            pltpu.VMEM((D, d_ff), jnp.bfloat16),
            pltpu.VMEM((d_ff, D), jnp.bfloat16),
            pltpu.VMEM((D, D), jnp.bfloat16),
        ],
        compiler_params=pltpu.CompilerParams(
            dimension_semantics=("arbitrary",)),
    )(hidden_states, W_pre, W_res, wrt, W1, W2, W_post)
    return out
